# baseline (speedup 1.0000x reference)
import functools

import jax
import jax.numpy as jnp
from jax import lax
from jax.experimental import pallas as pl
from jax.experimental.pallas import tpu as pltpu
from jax.experimental.pallas import tpu_sc as plsc

N = 10000
D = 128
E = 320000
NC = 2
NS = 16
NW = NC * NS
CHUNK = 128
EPW = 10240
NCH = EPW // CHUNK
BLK = 8
NBLK = NCH // BLK
NSLOT = 3
E_PAD = NW * EPW
NPAD = 10240
ROWS_PER_TILE = NPAD // NS


_ROW_BLK = 1000


def _mm_body(x_ref, w_ref, o_ref):
    o_ref[...] = jnp.dot(x_ref[...], w_ref[...],
                         preferred_element_type=jnp.float32)


def _matmul(x, w):
    return pl.pallas_call(
        _mm_body,
        grid=(N // _ROW_BLK,),
        in_specs=[
            pl.BlockSpec((_ROW_BLK, D), lambda i: (i, 0)),
            pl.BlockSpec((D, D), lambda i: (0, 0)),
        ],
        out_specs=pl.BlockSpec((_ROW_BLK, D), lambda i: (i, 0)),
        out_shape=jax.ShapeDtypeStruct((N, D), jnp.float32),
    )(x, w)


def _mid_body(p0_ref, p1_ref, b_ref, w_ref, o_ref):
    h = jnp.maximum(p0_ref[...] + p1_ref[...] + b_ref[...], 0.0)
    o_ref[...] = jnp.dot(h, w_ref[...], preferred_element_type=jnp.float32)


def _mid_layer(p0, p1, b, w):
    return pl.pallas_call(
        _mid_body,
        grid=(N // _ROW_BLK,),
        in_specs=[
            pl.BlockSpec((_ROW_BLK, D), lambda i: (i, 0)),
            pl.BlockSpec((_ROW_BLK, D), lambda i: (i, 0)),
            pl.BlockSpec((D,), lambda i: (0,)),
            pl.BlockSpec((D, D), lambda i: (0, 0)),
        ],
        out_specs=pl.BlockSpec((_ROW_BLK, D), lambda i: (i, 0)),
        out_shape=jax.ShapeDtypeStruct((N, D), jnp.float32),
    )(p0, p1, b, w)


def _comb_body(p0_ref, p1_ref, b_ref, o_ref):
    o_ref[...] = p0_ref[...] + p1_ref[...] + b_ref[...]


def _combine(p0, p1, b):
    return pl.pallas_call(
        _comb_body,
        grid=(N // _ROW_BLK,),
        in_specs=[
            pl.BlockSpec((_ROW_BLK, D), lambda i: (i, 0)),
            pl.BlockSpec((_ROW_BLK, D), lambda i: (i, 0)),
            pl.BlockSpec((D,), lambda i: (0,)),
        ],
        out_specs=pl.BlockSpec((_ROW_BLK, D), lambda i: (i, 0)),
        out_shape=jax.ShapeDtypeStruct((N, D), jnp.float32),
    )(p0, p1, b)


def _spmm_body(sup_hbm, src_hbm, dst_hbm, val_hbm, zeros_hbm,
               out_hbm, msg_hbm,
               src_v, dst_v, val_v, rows_v, shared, gsem, ssem, esem):
    c = lax.axis_index("c")
    s = lax.axis_index("s")
    w = c * NS + s

    def stage(ref_hbm, ref_v, B, slot, sync=False):
        if sync:
            pltpu.sync_copy(ref_hbm.at[w, B], ref_v.at[slot])
        else:
            pltpu.async_copy(ref_hbm.at[w, B], ref_v.at[slot], esem)

    def wait_stage(ref_hbm, ref_v, B, slot):
        pltpu.make_async_copy(ref_hbm.at[w, B], ref_v.at[slot], esem).wait()


    def start_gather(g, b):
        slot = lax.rem(g // BLK, NSLOT)
        pltpu.async_copy(shared.at[src_v.at[slot, g % BLK]], rows_v.at[b],
                         gsem)

    def wait_gather(g, b):
        slot = lax.rem(g // BLK, NSLOT)
        pltpu.make_async_copy(shared.at[src_v.at[slot, g % BLK]],
                              rows_v.at[b], gsem).wait()

    def start_msgwrite(g, b):
        pltpu.async_copy(rows_v.at[b], msg_hbm.at[w, g], ssem)

    def wait_msgwrite(g, b):
        pltpu.make_async_copy(rows_v.at[b], msg_hbm.at[w, g], ssem).wait()

    stage(src_hbm, src_v, 1, 1)
    stage(val_hbm, val_v, 1, 1)
    stage(src_hbm, src_v, 2, 2)
    stage(val_hbm, val_v, 2, 2)
    stage(src_hbm, src_v, 0, 0, sync=True)
    stage(val_hbm, val_v, 0, 0, sync=True)
    pltpu.sync_copy(sup_hbm.at[pl.ds(s * ROWS_PER_TILE, ROWS_PER_TILE)],
                    shared.at[pl.ds(s * ROWS_PER_TILE, ROWS_PER_TILE)])
    plsc.subcore_barrier()

    start_gather(0, 0)

    def p1_step(g, carry):
        b = lax.rem(g, 2)
        nb = 1 - b
        blk = g // BLK

        @pl.when(g >= 1)
        def _():
            wait_msgwrite(g - 1, nb)

        @pl.when((g % BLK == 0) & (blk >= 1) & (blk + 2 < NBLK))
        def _():
            slot = lax.rem(blk + 2, NSLOT)
            stage(src_hbm, src_v, blk + 2, slot)
            stage(val_hbm, val_v, blk + 2, slot)

        @pl.when(((g + 1) % BLK == 0) & (g + 1 < NCH))
        def _():
            nblk = (g + 1) // BLK
            slot = lax.rem(nblk, NSLOT)
            wait_stage(src_hbm, src_v, nblk, slot)
            wait_stage(val_hbm, val_v, nblk, slot)

        @pl.when(g + 1 < NCH)
        def _():
            start_gather(g + 1, nb)

        wait_gather(g, b)

        slot = lax.rem(blk, NSLOT)
        r = g % BLK

        def scale_group(eg, carry2):
            vv = val_v[slot, r, pl.ds(eg * 16, 16)]
            vs = [vv[k] for k in range(16)]
            for f in range(D // 16):
                sl = pl.ds(f * 16, 16)
                for k in range(16):
                    e = eg * 16 + k
                    rows_v[b, e, sl] = rows_v[b, e, sl] * vs[k]
            return carry2

        lax.fori_loop(0, CHUNK // 16, scale_group, 0)

        start_msgwrite(g, b)
        return carry

    lax.fori_loop(0, NCH, p1_step, 0)
    wait_msgwrite(NCH - 1, (NCH - 1) % 2)

    plsc.subcore_barrier()


    def start_msgread(g, b):
        pltpu.async_copy(msg_hbm.at[w, g], rows_v.at[b], gsem)

    def wait_msgread(g, b):
        pltpu.make_async_copy(msg_hbm.at[w, g], rows_v.at[b], gsem).wait()

    def start_scatter(g, b):
        slot = lax.rem(g // BLK, NSLOT)
        pltpu.async_copy(rows_v.at[b], shared.at[dst_v.at[slot, g % BLK]],
                         ssem, add=True)

    def wait_scatter(g, b):
        slot = lax.rem(g // BLK, NSLOT)
        pltpu.make_async_copy(rows_v.at[b], shared.at[dst_v.at[slot, g % BLK]],
                              ssem).wait()

    stage(dst_hbm, dst_v, 1, 1)
    stage(dst_hbm, dst_v, 2, 2)
    stage(dst_hbm, dst_v, 0, 0, sync=True)
    pltpu.sync_copy(zeros_hbm,
                    shared.at[pl.ds(s * ROWS_PER_TILE, ROWS_PER_TILE)])
    plsc.subcore_barrier()

    start_msgread(0, 0)

    def p2_step(g, carry):
        b = lax.rem(g, 2)
        nb = 1 - b
        blk = g // BLK

        @pl.when(g >= 1)
        def _():
            wait_scatter(g - 1, nb)

        @pl.when((g % BLK == 0) & (blk >= 1) & (blk + 2 < NBLK))
        def _():
            stage(dst_hbm, dst_v, blk + 2, lax.rem(blk + 2, NSLOT))

        @pl.when((g % BLK == 0) & (blk >= 1))
        def _():
            wait_stage(dst_hbm, dst_v, blk, lax.rem(blk, NSLOT))

        @pl.when(g + 1 < NCH)
        def _():
            start_msgread(g + 1, nb)

        wait_msgread(g, b)
        start_scatter(g, b)
        return carry

    lax.fori_loop(0, NCH, p2_step, 0)
    wait_scatter(NCH - 1, (NCH - 1) % 2)

    plsc.subcore_barrier()
    pltpu.sync_copy(shared.at[pl.ds(s * ROWS_PER_TILE, ROWS_PER_TILE)],
                    out_hbm.at[c, pl.ds(s * ROWS_PER_TILE, ROWS_PER_TILE)])


_spmm_call = pl.kernel(
    _spmm_body,
    out_type=[jax.ShapeDtypeStruct((NC, NPAD, D), jnp.float32),
              jax.ShapeDtypeStruct((NW, NCH, CHUNK, D), jnp.float32)],
    mesh=plsc.VectorSubcoreMesh(core_axis_name="c", subcore_axis_name="s"),
    scratch_types=[
        pltpu.VMEM((NSLOT, BLK, CHUNK), jnp.int32),
        pltpu.VMEM((NSLOT, BLK, CHUNK), jnp.int32),
        pltpu.VMEM((NSLOT, BLK, CHUNK), jnp.float32),
        pltpu.VMEM((2, CHUNK, D), jnp.float32),
        pltpu.VMEM_SHARED((NPAD, D), jnp.float32),
        pltpu.SemaphoreType.DMA,
        pltpu.SemaphoreType.DMA,
        pltpu.SemaphoreType.DMA,
    ],
)


def kernel(x, adj_indices, adj_values, W1, b1, W2, b2):
    dst = adj_indices[0].astype(jnp.int32)
    src = adj_indices[1].astype(jnp.int32)
    val = adj_values.astype(jnp.float32)

    pad = E_PAD - E
    src3 = jnp.pad(src, (0, pad)).reshape(NW, NBLK, BLK, CHUNK)
    dst3 = jnp.pad(dst, (0, pad)).reshape(NW, NBLK, BLK, CHUNK)
    val3 = jnp.pad(val, (0, pad)).reshape(NW, NBLK, BLK, CHUNK)
    zeros = jnp.zeros((ROWS_PER_TILE, D), jnp.float32)

    sup1 = jnp.pad(_matmul(x, W1), ((0, NPAD - N), (0, 0)))
    parts1, _ = _spmm_call(sup1, src3, dst3, val3, zeros)
    sup2 = jnp.pad(_mid_layer(parts1[0], parts1[1], b1, W2),
                   ((0, NPAD - N), (0, 0)))
    parts2, _ = _spmm_call(sup2, src3, dst3, val3, zeros)
    return _combine(parts2[0], parts2[1], b2)


# speedup vs baseline: 1.0390x; 1.0390x over previous
import functools

import jax
import jax.numpy as jnp
from jax import lax
from jax.experimental import pallas as pl
from jax.experimental.pallas import tpu as pltpu
from jax.experimental.pallas import tpu_sc as plsc

N = 10000
D = 128
E = 320000
NC = 2
NS = 16
NW = NC * NS
CHUNK = 128
EPW = 10240
NCH = EPW // CHUNK
BLK = 8
NBLK = NCH // BLK
NSLOT = 3
E_PAD = NW * EPW
NPAD = 10240
ROWS_PER_TILE = NPAD // NS


_ROW_BLK = 1000


def _mm_body(x_ref, w_ref, o_ref):
    o_ref[...] = jnp.dot(x_ref[...], w_ref[...],
                         preferred_element_type=jnp.float32)


def _matmul(x, w):
    return pl.pallas_call(
        _mm_body,
        grid=(N // _ROW_BLK,),
        in_specs=[
            pl.BlockSpec((_ROW_BLK, D), lambda i: (i, 0)),
            pl.BlockSpec((D, D), lambda i: (0, 0)),
        ],
        out_specs=pl.BlockSpec((_ROW_BLK, D), lambda i: (i, 0)),
        out_shape=jax.ShapeDtypeStruct((NPAD, D), jnp.float32),
    )(x, w)


def _mid_body(p_ref, b_ref, w_ref, o_ref):
    h = jnp.maximum(p_ref[0] + p_ref[1] + b_ref[...], 0.0)
    o_ref[...] = jnp.dot(h, w_ref[...], preferred_element_type=jnp.float32)


def _mid_layer(parts, b, w):
    return pl.pallas_call(
        _mid_body,
        grid=(N // _ROW_BLK,),
        in_specs=[
            pl.BlockSpec((NC, _ROW_BLK, D), lambda i: (0, i, 0)),
            pl.BlockSpec((D,), lambda i: (0,)),
            pl.BlockSpec((D, D), lambda i: (0, 0)),
        ],
        out_specs=pl.BlockSpec((_ROW_BLK, D), lambda i: (i, 0)),
        out_shape=jax.ShapeDtypeStruct((NPAD, D), jnp.float32),
    )(parts, b, w)


def _comb_body(p_ref, b_ref, o_ref):
    o_ref[...] = p_ref[0] + p_ref[1] + b_ref[...]


def _combine(parts, b):
    return pl.pallas_call(
        _comb_body,
        grid=(N // _ROW_BLK,),
        in_specs=[
            pl.BlockSpec((NC, _ROW_BLK, D), lambda i: (0, i, 0)),
            pl.BlockSpec((D,), lambda i: (0,)),
        ],
        out_specs=pl.BlockSpec((_ROW_BLK, D), lambda i: (i, 0)),
        out_shape=jax.ShapeDtypeStruct((N, D), jnp.float32),
    )(parts, b)


def _spmm_body(sup_hbm, src_hbm, dst_hbm, val_hbm, zeros_hbm,
               out_hbm, msg_hbm,
               src_v, dst_v, val_v, rows_v, shared, gsem, ssem, esem):
    c = lax.axis_index("c")
    s = lax.axis_index("s")
    w = c * NS + s

    def stage(ref_hbm, ref_v, B, slot, sync=False):
        if sync:
            pltpu.sync_copy(ref_hbm.at[w, B], ref_v.at[slot])
        else:
            pltpu.async_copy(ref_hbm.at[w, B], ref_v.at[slot], esem)

    def wait_stage(ref_hbm, ref_v, B, slot):
        pltpu.make_async_copy(ref_hbm.at[w, B], ref_v.at[slot], esem).wait()


    def start_gather(g, b):
        slot = lax.rem(g // BLK, NSLOT)
        pltpu.async_copy(shared.at[src_v.at[slot, g % BLK]], rows_v.at[b],
                         gsem)

    def wait_gather(g, b):
        slot = lax.rem(g // BLK, NSLOT)
        pltpu.make_async_copy(shared.at[src_v.at[slot, g % BLK]],
                              rows_v.at[b], gsem).wait()

    def start_msgwrite(g, b):
        pltpu.async_copy(rows_v.at[b], msg_hbm.at[w, g], ssem)

    def wait_msgwrite(g, b):
        pltpu.make_async_copy(rows_v.at[b], msg_hbm.at[w, g], ssem).wait()

    stage(src_hbm, src_v, 1, 1)
    stage(val_hbm, val_v, 1, 1)
    stage(src_hbm, src_v, 2, 2)
    stage(val_hbm, val_v, 2, 2)
    stage(src_hbm, src_v, 0, 0, sync=True)
    stage(val_hbm, val_v, 0, 0, sync=True)
    pltpu.sync_copy(sup_hbm.at[pl.ds(s * ROWS_PER_TILE, ROWS_PER_TILE)],
                    shared.at[pl.ds(s * ROWS_PER_TILE, ROWS_PER_TILE)])
    plsc.subcore_barrier()

    start_gather(0, 0)

    def p1_step(g, carry):
        b = lax.rem(g, 2)
        nb = 1 - b
        blk = g // BLK

        @pl.when(g >= 1)
        def _():
            wait_msgwrite(g - 1, nb)

        @pl.when((g % BLK == 0) & (blk >= 1) & (blk + 2 < NBLK))
        def _():
            slot = lax.rem(blk + 2, NSLOT)
            stage(src_hbm, src_v, blk + 2, slot)
            stage(val_hbm, val_v, blk + 2, slot)

        @pl.when(((g + 1) % BLK == 0) & (g + 1 < NCH))
        def _():
            nblk = (g + 1) // BLK
            slot = lax.rem(nblk, NSLOT)
            wait_stage(src_hbm, src_v, nblk, slot)
            wait_stage(val_hbm, val_v, nblk, slot)

        @pl.when(g + 1 < NCH)
        def _():
            start_gather(g + 1, nb)

        wait_gather(g, b)

        slot = lax.rem(blk, NSLOT)
        r = g % BLK

        def scale_group(eg, carry2):
            vv = val_v[slot, r, pl.ds(eg * 16, 16)]
            vs = [vv[k] for k in range(16)]
            for f in range(D // 16):
                sl = pl.ds(f * 16, 16)
                for k in range(16):
                    e = eg * 16 + k
                    rows_v[b, e, sl] = rows_v[b, e, sl] * vs[k]
            return carry2

        lax.fori_loop(0, CHUNK // 16, scale_group, 0)

        start_msgwrite(g, b)
        return carry

    lax.fori_loop(0, NCH, p1_step, 0)
    wait_msgwrite(NCH - 1, (NCH - 1) % 2)

    plsc.subcore_barrier()


    def start_msgread(g, b):
        pltpu.async_copy(msg_hbm.at[w, g], rows_v.at[b], gsem)

    def wait_msgread(g, b):
        pltpu.make_async_copy(msg_hbm.at[w, g], rows_v.at[b], gsem).wait()

    def start_scatter(g, b):
        slot = lax.rem(g // BLK, NSLOT)
        pltpu.async_copy(rows_v.at[b], shared.at[dst_v.at[slot, g % BLK]],
                         ssem, add=True)

    def wait_scatter(g, b):
        slot = lax.rem(g // BLK, NSLOT)
        pltpu.make_async_copy(rows_v.at[b], shared.at[dst_v.at[slot, g % BLK]],
                              ssem).wait()

    stage(dst_hbm, dst_v, 1, 1)
    stage(dst_hbm, dst_v, 2, 2)
    stage(dst_hbm, dst_v, 0, 0, sync=True)
    pltpu.sync_copy(zeros_hbm,
                    shared.at[pl.ds(s * ROWS_PER_TILE, ROWS_PER_TILE)])
    plsc.subcore_barrier()

    start_msgread(0, 0)

    def p2_step(g, carry):
        b = lax.rem(g, 2)
        nb = 1 - b
        blk = g // BLK

        @pl.when(g >= 1)
        def _():
            wait_scatter(g - 1, nb)

        @pl.when((g % BLK == 0) & (blk >= 1) & (blk + 2 < NBLK))
        def _():
            stage(dst_hbm, dst_v, blk + 2, lax.rem(blk + 2, NSLOT))

        @pl.when((g % BLK == 0) & (blk >= 1))
        def _():
            wait_stage(dst_hbm, dst_v, blk, lax.rem(blk, NSLOT))

        @pl.when(g + 1 < NCH)
        def _():
            start_msgread(g + 1, nb)

        wait_msgread(g, b)
        start_scatter(g, b)
        return carry

    lax.fori_loop(0, NCH, p2_step, 0)
    wait_scatter(NCH - 1, (NCH - 1) % 2)

    plsc.subcore_barrier()
    pltpu.sync_copy(shared.at[pl.ds(s * ROWS_PER_TILE, ROWS_PER_TILE)],
                    out_hbm.at[c, pl.ds(s * ROWS_PER_TILE, ROWS_PER_TILE)])


_spmm_call = pl.kernel(
    _spmm_body,
    out_type=[jax.ShapeDtypeStruct((NC, NPAD, D), jnp.float32),
              jax.ShapeDtypeStruct((NW, NCH, CHUNK, D), jnp.float32)],
    mesh=plsc.VectorSubcoreMesh(core_axis_name="c", subcore_axis_name="s"),
    scratch_types=[
        pltpu.VMEM((NSLOT, BLK, CHUNK), jnp.int32),
        pltpu.VMEM((NSLOT, BLK, CHUNK), jnp.int32),
        pltpu.VMEM((NSLOT, BLK, CHUNK), jnp.float32),
        pltpu.VMEM((2, CHUNK, D), jnp.float32),
        pltpu.VMEM_SHARED((NPAD, D), jnp.float32),
        pltpu.SemaphoreType.DMA,
        pltpu.SemaphoreType.DMA,
        pltpu.SemaphoreType.DMA,
    ],
)


def kernel(x, adj_indices, adj_values, W1, b1, W2, b2):
    dst = adj_indices[0].astype(jnp.int32)
    src = adj_indices[1].astype(jnp.int32)
    val = adj_values.astype(jnp.float32)

    pad = E_PAD - E
    src3 = jnp.pad(src, (0, pad)).reshape(NW, NBLK, BLK, CHUNK)
    dst3 = jnp.pad(dst, (0, pad)).reshape(NW, NBLK, BLK, CHUNK)
    val3 = jnp.pad(val, (0, pad)).reshape(NW, NBLK, BLK, CHUNK)
    zeros = jnp.zeros((ROWS_PER_TILE, D), jnp.float32)

    sup1 = _matmul(x, W1)
    parts1, _ = _spmm_call(sup1, src3, dst3, val3, zeros)
    sup2 = _mid_layer(parts1, b1, W2)
    parts2, _ = _spmm_call(sup2, src3, dst3, val3, zeros)
    return _combine(parts2, b2)
